# baseline (speedup 1.0000x reference)
import functools

import jax
import jax.numpy as jnp
import numpy as np
from jax.experimental import pallas as pl
from jax.experimental.pallas import tpu as pltpu

N_ENT = 10000
N_REL = 237
N_EDGE = 320000
H_DIM = 128
BS = 1024
OUT_CH = 32
KER = 7
K_H = 8
K_W = 16
FLAT = OUT_CH * (2 * K_H - KER + 1) * (K_W - KER + 1)
N_POS = (2 * K_H - KER + 1) * (K_W - KER + 1)
IMG = 2 * K_H * K_W

N_ENT_PAD = 10240


def _conv_as_matmul_indices():
    o, i, j, kh, kw = np.meshgrid(
        np.arange(OUT_CH), np.arange(10), np.arange(10),
        np.arange(KER), np.arange(KER), indexing="ij")
    rows = (K_W * (i + kh) + (j + kw)).ravel()
    cols = (o * N_POS + i * 10 + j).ravel()
    vals_o = o.ravel()
    vals_kh = kh.ravel()
    vals_kw = kw.ravel()
    return rows, cols, vals_o, vals_kh, vals_kw


_ROWS, _COLS, _VO, _VKH, _VKW = _conv_as_matmul_indices()


def _edge_softmax(scores, dst, n):
    m = jax.ops.segment_max(scores, dst, num_segments=n)
    m = jnp.where(jnp.isfinite(m), m, 0.0)
    ex = jnp.exp(scores - m[dst])
    den = jax.ops.segment_sum(ex, dst, num_segments=n)
    return ex / jnp.maximum(den[dst], 1e-16)


def _message_passing(ent_emb, rel_emb_1, rel_emb_2, W_edge_1, W_node_1,
                     W_comp_1, W_edge_2, W_node_2, W_comp_2,
                     edge_index, rel_id):
    src = edge_index[0]
    dst = edge_index[1]
    n = ent_emb.shape[0]
    x = ent_emb
    for rel_emb, We, Wn, Wc in ((rel_emb_1, W_edge_1, W_node_1, W_comp_1),
                                (rel_emb_2, W_edge_2, W_node_2, W_comp_2)):
        h_src = x[src]
        h_dst = x[dst]
        r_e = rel_emb[rel_id]
        a = _edge_softmax(jnp.sum(r_e * h_dst, axis=-1), dst, n)
        edge_out = jnp.tanh(jax.ops.segment_sum(r_e * a[:, None], dst, num_segments=n) @ We)
        a = _edge_softmax(jnp.sum(h_src * h_dst, axis=-1), dst, n)
        node_out = jnp.tanh(jax.ops.segment_sum(h_src * a[:, None], dst, num_segments=n) @ Wn)
        comp = h_src * r_e
        a = _edge_softmax(jnp.sum(comp * h_dst, axis=-1), dst, n)
        comp_out = jnp.tanh(jax.ops.segment_sum(comp * a[:, None], dst, num_segments=n) @ Wc)
        x = x + edge_out + node_out + comp_out
    return x


def _mlp_kernel(img_ref, m_ref, mb_ref, fcw_ref, fcb_ref, out_ref):
    y1 = jnp.maximum(
        jnp.dot(img_ref[...], m_ref[...], preferred_element_type=jnp.float32)
        + mb_ref[...], 0.0)
    y2 = jnp.maximum(
        jnp.dot(y1, fcw_ref[...], preferred_element_type=jnp.float32)
        + fcb_ref[...], 0.0)
    out_ref[...] = y2


def _logits_kernel(y_ref, x_ref, b_ref, out_ref):
    l = jax.lax.dot_general(
        y_ref[...], x_ref[...],
        (((1,), (1,)), ((), ())), preferred_element_type=jnp.float32)
    out_ref[...] = jax.nn.sigmoid(l + b_ref[...])


def _predictor(x, h_id, r_id, pred_rel_emb, conv_w, conv_b, fc_w, fc_b,
               ent_bias):
    head = x[h_id]
    rel = pred_rel_emb[r_id]
    img = jnp.stack([head, rel], axis=2).reshape(BS, IMG)

    vals = conv_w[_VO, 0, _VKH, _VKW]
    M = jnp.zeros((IMG, FLAT), jnp.float32).at[_ROWS, _COLS].add(vals)
    mb = jnp.repeat(conv_b, N_POS).reshape(1, FLAT)

    y2 = pl.pallas_call(
        _mlp_kernel,
        out_shape=jax.ShapeDtypeStruct((BS, H_DIM), jnp.float32),
    )(img, M, mb, fc_w, fc_b.reshape(1, H_DIM))

    x_pad = jnp.zeros((N_ENT_PAD, H_DIM), jnp.float32).at[:N_ENT].set(x)
    bias_pad = jnp.zeros((1, N_ENT_PAD), jnp.float32).at[0, :N_ENT].set(ent_bias)

    blk = 2048
    probs = pl.pallas_call(
        _logits_kernel,
        grid=(N_ENT_PAD // blk,),
        in_specs=[
            pl.BlockSpec((BS, H_DIM), lambda i: (0, 0)),
            pl.BlockSpec((blk, H_DIM), lambda i: (i, 0)),
            pl.BlockSpec((1, blk), lambda i: (0, i)),
        ],
        out_specs=pl.BlockSpec((BS, blk), lambda i: (0, i)),
        out_shape=jax.ShapeDtypeStruct((BS, N_ENT_PAD), jnp.float32),
    )(y2, x_pad, bias_pad)
    return probs[:, :N_ENT]


def kernel(h_id, r_id, edge_index, rel_id, ent_emb, rel_emb_1, rel_emb_2,
           W_edge_1, W_node_1, W_comp_1, W_edge_2, W_node_2, W_comp_2,
           pred_rel_emb, conv_w, conv_b, fc_w, fc_b, ent_bias):
    x = _message_passing(ent_emb, rel_emb_1, rel_emb_2, W_edge_1, W_node_1,
                         W_comp_1, W_edge_2, W_node_2, W_comp_2,
                         edge_index, rel_id)
    return _predictor(x, h_id, r_id, pred_rel_emb, conv_w, conv_b,
                      fc_w, fc_b, ent_bias)


# speedup vs baseline: 7.8985x; 7.8985x over previous
import functools

import jax
import jax.numpy as jnp
import numpy as np
from jax import lax
from jax.experimental import pallas as pl
from jax.experimental.pallas import tpu as pltpu
from jax.experimental.pallas import tpu_sc as plsc

N_ENT = 10000
N_REL = 237
N_EDGE = 320000
H_DIM = 128
BS = 1024
OUT_CH = 32
KER = 7
K_H = 8
K_W = 16
FLAT = OUT_CH * (2 * K_H - KER + 1) * (K_W - KER + 1)
N_POS = (2 * K_H - KER + 1) * (K_W - KER + 1)
IMG = 2 * K_H * K_W
N_RELS = 2 * N_REL
N_RELS_PAD = 480

N_ENT_PAD = 10240
X_ROWS = 16384
NW = 32
CHUNK = 128
N_CHUNKS = N_EDGE // CHUNK
CHUNKS_PER_W = -(-N_CHUNKS // NW)
CHUNK2 = 64
N_CHUNKS2 = N_EDGE // CHUNK2
CHUNKS2_PER_W = -(-N_CHUNKS2 // NW)
ENT_SLICE = N_ENT_PAD // 16

_MESH = dict(core_axis_name="c", subcore_axis_name="s")

f32 = jnp.float32
i32 = jnp.int32


def _iota16():
    return lax.broadcasted_iota(i32, (16,), 0)


def _pass1_body(x_hbm, rel_hbm, ei_hbm, rid_hbm,
                ex_e_hbm, ex_n_hbm, ex_c_hbm,
                den_e_hbm, den_n_hbm, den_c_hbm,
                rel_loc, hdst, hsrc, dstv, srcv, relv,
                exe_v, exn_v, exc_v, zb,
                den_e_sh, den_n_sh, den_c_sh, sem1, sem2):
    c = lax.axis_index("c")
    s = lax.axis_index("s")
    wid = s * 2 + c
    base = s * ENT_SLICE

    def zb_body(i, _):
        zb[pl.ds(i * 16, 16)] = jnp.zeros((16,), f32)
        return 0
    lax.fori_loop(0, ENT_SLICE // 16, zb_body, 0)
    pltpu.sync_copy(zb, den_e_sh.at[pl.ds(base, ENT_SLICE)])
    pltpu.sync_copy(zb, den_n_sh.at[pl.ds(base, ENT_SLICE)])
    pltpu.sync_copy(zb, den_c_sh.at[pl.ds(base, ENT_SLICE)])
    pltpu.sync_copy(rel_hbm, rel_loc)
    plsc.subcore_barrier()

    def chunk_body(k, _):
        cidx = k * NW + wid

        @pl.when(cidx < N_CHUNKS)
        def _():
            off = cidx * CHUNK
            pltpu.sync_copy(ei_hbm.at[1, pl.ds(off, CHUNK)], dstv)
            pltpu.sync_copy(ei_hbm.at[0, pl.ds(off, CHUNK)], srcv)
            pltpu.sync_copy(rid_hbm.at[pl.ds(off, CHUNK)], relv)
            cp1 = pltpu.async_copy(x_hbm.at[dstv], hdst, sem1)
            cp2 = pltpu.async_copy(x_hbm.at[srcv], hsrc, sem2)
            cp1.wait()
            cp2.wait()

            def g_body(g, _):
                gb = g * 16
                rel16 = relv[pl.ds(gb, 16)]
                se = sn = sc = jnp.zeros((16,), f32)
                for lane in range(16):
                    e = gb + lane
                    r = rel16[lane]
                    ae = an = ac = jnp.zeros((16,), f32)
                    for k2 in range(H_DIM // 16):
                        o = k2 * 16
                        d = hdst[e, pl.ds(o, 16)]
                        s_ = hsrc[e, pl.ds(o, 16)]
                        re_ = rel_loc[r, pl.ds(o, 16)]
                        ae = ae + re_ * d
                        an = an + s_ * d
                        ac = ac + s_ * re_ * d
                    lm = _iota16() == lane
                    se = jnp.where(lm, jnp.sum(ae), se)
                    sn = jnp.where(lm, jnp.sum(an), sn)
                    sc = jnp.where(lm, jnp.sum(ac), sc)
                exe_v[pl.ds(gb, 16)] = jnp.exp(se)
                exn_v[pl.ds(gb, 16)] = jnp.exp(sn)
                exc_v[pl.ds(gb, 16)] = jnp.exp(sc)
                return 0

            lax.fori_loop(0, CHUNK // 16, g_body, 0)
            pltpu.sync_copy(exe_v, ex_e_hbm.at[cidx])
            pltpu.sync_copy(exn_v, ex_n_hbm.at[cidx])
            pltpu.sync_copy(exc_v, ex_c_hbm.at[cidx])
            pltpu.sync_copy(exe_v, den_e_sh.at[dstv], add=True)
            pltpu.sync_copy(exn_v, den_n_sh.at[dstv], add=True)
            pltpu.sync_copy(exc_v, den_c_sh.at[dstv], add=True)
        return 0

    lax.fori_loop(0, CHUNKS_PER_W, chunk_body, 0)
    plsc.subcore_barrier()
    ob = c * N_ENT_PAD + base
    pltpu.sync_copy(den_e_sh.at[pl.ds(base, ENT_SLICE)],
                    den_e_hbm.at[pl.ds(ob, ENT_SLICE)])
    pltpu.sync_copy(den_n_sh.at[pl.ds(base, ENT_SLICE)],
                    den_n_hbm.at[pl.ds(ob, ENT_SLICE)])
    pltpu.sync_copy(den_c_sh.at[pl.ds(base, ENT_SLICE)],
                    den_c_hbm.at[pl.ds(ob, ENT_SLICE)])


def _pass1(x_pad, rel_tbl, ei, rid):
    out_type = (
        [jax.ShapeDtypeStruct((N_CHUNKS, CHUNK), f32)] * 3
        + [jax.ShapeDtypeStruct((2 * N_ENT_PAD,), f32)] * 3)
    scratch = [
        pltpu.VMEM((N_RELS_PAD, H_DIM), f32),
        pltpu.VMEM((CHUNK, H_DIM), f32),
        pltpu.VMEM((CHUNK, H_DIM), f32),
        pltpu.VMEM((CHUNK,), i32),
        pltpu.VMEM((CHUNK,), i32),
        pltpu.VMEM((CHUNK,), i32),
        pltpu.VMEM((CHUNK,), f32),
        pltpu.VMEM((CHUNK,), f32),
        pltpu.VMEM((CHUNK,), f32),
        pltpu.VMEM((ENT_SLICE,), f32),
        pltpu.VMEM_SHARED((N_ENT_PAD,), f32),
        pltpu.VMEM_SHARED((N_ENT_PAD,), f32),
        pltpu.VMEM_SHARED((N_ENT_PAD,), f32),
        pltpu.SemaphoreType.DMA,
        pltpu.SemaphoreType.DMA,
    ]
    return pl.kernel(
        _pass1_body, out_type=out_type,
        mesh=plsc.VectorSubcoreMesh(**_MESH),
        compiler_params=pltpu.CompilerParams(needs_layout_passes=False),
        scratch_types=scratch,
    )(x_pad, rel_tbl, ei, rid)


def _pass2_body(x_hbm, rel_hbm, ei_hbm, rid_hbm,
                exe_hbm, exn_hbm, exc_hbm,
                dene_hbm, denn_hbm, denc_hbm,
                oute_hbm, outn_hbm, outc_hbm,
                re_b, hs, upd, dstv, srcv, relv, exv, denv, zb,
                acc_sh, sem1, sem2):
    c = lax.axis_index("c")
    s = lax.axis_index("s")
    wid = s * 2 + c
    base = s * ENT_SLICE

    for r in range(16):
        def zb_body(j, _):
            zb[r, pl.ds(j * 16, 16)] = jnp.zeros((16,), f32)
            return 0
        lax.fori_loop(0, H_DIM // 16, zb_body, 0)

    for mode, ex_hbm, den_hbm, out_hbm in (
            (0, exe_hbm, dene_hbm, oute_hbm),
            (1, exn_hbm, denn_hbm, outn_hbm),
            (2, exc_hbm, denc_hbm, outc_hbm)):
        need_rel = mode in (0, 2)
        need_src = mode in (1, 2)
        for t in range(ENT_SLICE // 16):
            pltpu.sync_copy(zb, acc_sh.at[pl.ds(base + t * 16, 16)])
        pltpu.sync_copy(den_hbm, denv)
        plsc.subcore_barrier()

        def chunk_body(k, _):
            cidx = k * NW + wid

            @pl.when(cidx < N_CHUNKS2)
            def _():
                off = cidx * CHUNK2
                pltpu.sync_copy(ei_hbm.at[1, pl.ds(off, CHUNK2)], dstv)
                pltpu.sync_copy(ex_hbm.at[cidx], exv)
                if need_rel:
                    pltpu.sync_copy(rid_hbm.at[pl.ds(off, CHUNK2)], relv)
                    cp2 = pltpu.async_copy(rel_hbm.at[relv], re_b, sem2)
                if need_src:
                    pltpu.sync_copy(ei_hbm.at[0, pl.ds(off, CHUNK2)], srcv)
                    cp1 = pltpu.async_copy(x_hbm.at[srcv], hs, sem1)
                if need_rel:
                    cp2.wait()
                if need_src:
                    cp1.wait()

                def g_body(g, _):
                    gb = g * 16
                    dst16 = dstv[pl.ds(gb, 16)]
                    d16 = plsc.load_gather(denv, [dst16])
                    a16 = exv[pl.ds(gb, 16)] / d16
                    for lane in range(16):
                        e = gb + lane
                        a_s = a16[lane]
                        for k2 in range(H_DIM // 16):
                            o = k2 * 16
                            if mode == 0:
                                m = re_b[e, pl.ds(o, 16)]
                            elif mode == 1:
                                m = hs[e, pl.ds(o, 16)]
                            else:
                                m = re_b[e, pl.ds(o, 16)] * hs[e, pl.ds(o, 16)]
                            upd[e, pl.ds(o, 16)] = a_s * m
                    return 0

                lax.fori_loop(0, CHUNK2 // 16, g_body, 0)
                pltpu.sync_copy(upd, acc_sh.at[dstv], add=True)
            return 0

        lax.fori_loop(0, CHUNKS2_PER_W, chunk_body, 0)
        plsc.subcore_barrier()
        ob = c * N_ENT_PAD + base
        for t in range(ENT_SLICE // CHUNK):
            pltpu.sync_copy(acc_sh.at[pl.ds(base + t * CHUNK, CHUNK)],
                            out_hbm.at[pl.ds(ob + t * CHUNK, CHUNK)])


def _pass2(x_pad, rel_tbl, ei, rid, exe, exn, exc, dene, denn, denc):
    out_type = [jax.ShapeDtypeStruct((2 * N_ENT_PAD, H_DIM), f32)] * 3
    scratch = [
        pltpu.VMEM((CHUNK2, H_DIM), f32),
        pltpu.VMEM((CHUNK2, H_DIM), f32),
        pltpu.VMEM((CHUNK2, H_DIM), f32),
        pltpu.VMEM((CHUNK2,), i32),
        pltpu.VMEM((CHUNK2,), i32),
        pltpu.VMEM((CHUNK2,), i32),
        pltpu.VMEM((CHUNK2,), f32),
        pltpu.VMEM((N_ENT_PAD,), f32),
        pltpu.VMEM((16, H_DIM), f32),
        pltpu.VMEM_SHARED((N_ENT_PAD, H_DIM), f32),
        pltpu.SemaphoreType.DMA,
        pltpu.SemaphoreType.DMA,
    ]
    outs = pl.kernel(
        _pass2_body, out_type=out_type,
        mesh=plsc.VectorSubcoreMesh(**_MESH),
        compiler_params=pltpu.CompilerParams(needs_layout_passes=False),
        scratch_types=scratch,
    )(x_pad, rel_tbl, ei, rid, exe, exn, exc, dene, denn, denc)
    return tuple(o.reshape(2, N_ENT_PAD, H_DIM) for o in outs)


def _dencomb_kernel(d_ref, exe_ref, exn_ref, exc_ref, out_ref):
    out_ref[...] = (d_ref[:, 0, :] + d_ref[:, 1, :]
                    + exe_ref[0, 0] * 0.0 + exn_ref[0, 0] * 0.0
                    + exc_ref[0, 0] * 0.0)


def _den_combine(dene, denn, denc, exe, exn, exc):
    d = jnp.stack([dene, denn, denc]).reshape(3, 2, N_ENT_PAD)
    out = pl.pallas_call(
        _dencomb_kernel,
        out_shape=jax.ShapeDtypeStruct((3, N_ENT_PAD), f32),
    )(d, exe, exn, exc)
    return out[0], out[1], out[2]


def _combine_kernel(x_ref, ae_ref, an_ref, ac_ref, we_ref, wn_ref, wc_ref,
                    out_ref):
    ae = ae_ref[0] + ae_ref[1]
    an = an_ref[0] + an_ref[1]
    ac = ac_ref[0] + ac_ref[1]
    out_ref[...] = (
        x_ref[...]
        + jnp.tanh(jnp.dot(ae, we_ref[...], preferred_element_type=f32))
        + jnp.tanh(jnp.dot(an, wn_ref[...], preferred_element_type=f32))
        + jnp.tanh(jnp.dot(ac, wc_ref[...], preferred_element_type=f32)))


def _combine(x_pad, acc_e, acc_n, acc_c, We, Wn, Wc):
    blk = 1024
    acc_spec = pl.BlockSpec((2, blk, H_DIM), lambda i: (0, i, 0))
    w_spec = pl.BlockSpec((H_DIM, H_DIM), lambda i: (0, 0))
    return pl.pallas_call(
        _combine_kernel,
        grid=(N_ENT_PAD // blk,),
        in_specs=[pl.BlockSpec((blk, H_DIM), lambda i: (i, 0)),
                  acc_spec, acc_spec, acc_spec, w_spec, w_spec, w_spec],
        out_specs=pl.BlockSpec((blk, H_DIM), lambda i: (i, 0)),
        out_shape=jax.ShapeDtypeStruct((N_ENT_PAD, H_DIM), f32),
    )(x_pad, acc_e, acc_n, acc_c, We, Wn, Wc)


def _gather_body(tbl, ix, out, ixv, rows, sem):
    per = ixv.shape[0]
    wid = lax.axis_index("s") * 2 + lax.axis_index("c")
    base = wid * per
    pltpu.sync_copy(ix.at[pl.ds(base, per)], ixv)
    pltpu.async_copy(tbl.at[ixv], rows, sem).wait()
    pltpu.sync_copy(rows, out.at[pl.ds(base, per)])


def _gather_rows(table, idx):
    B = idx.shape[0]
    per = B // NW
    return pl.kernel(
        _gather_body,
        out_type=jax.ShapeDtypeStruct((B, table.shape[1]), f32),
        mesh=plsc.VectorSubcoreMesh(**_MESH),
        compiler_params=pltpu.CompilerParams(needs_layout_passes=False),
        scratch_types=[pltpu.VMEM((per,), i32),
                       pltpu.VMEM((per, table.shape[1]), f32),
                       pltpu.SemaphoreType.DMA],
    )(table, idx)


def _conv_as_matmul_tensor():
    T = np.zeros((IMG, N_POS, KER * KER), np.float32)
    for i in range(10):
        for j in range(10):
            for kh in range(KER):
                for kw in range(KER):
                    T[K_W * (i + kh) + (j + kw), i * 10 + j,
                      kh * KER + kw] = 1.0
    return T.reshape(IMG * N_POS, KER * KER)


_CONV_T = _conv_as_matmul_tensor()


def _mlp_kernel(img_ref, m_ref, mb_ref, fcw_ref, fcb_ref, out_ref):
    y1 = jnp.maximum(
        jnp.dot(img_ref[...], m_ref[...], preferred_element_type=f32)
        + mb_ref[...], 0.0)
    y2 = jnp.maximum(
        jnp.dot(y1, fcw_ref[...], preferred_element_type=f32)
        + fcb_ref[...], 0.0)
    out_ref[...] = y2


def _logits_kernel(y_ref, x_ref, b_ref, out_ref):
    l = lax.dot_general(y_ref[...], x_ref[...],
                        (((1,), (1,)), ((), ())), preferred_element_type=f32)
    out_ref[...] = jax.nn.sigmoid(l + b_ref[...])


def _predictor(x_big, h_id, r_id, pred_rel_emb, conv_w, conv_b, fc_w, fc_b,
               ent_bias):
    x_pad = x_big[:N_ENT_PAD]
    head = _gather_rows(x_big, h_id)
    rel = _gather_rows(pred_rel_emb, r_id)
    img = jnp.stack([head, rel], axis=2).reshape(BS, IMG)

    w2 = conv_w.reshape(OUT_CH, KER * KER)
    M = (jnp.asarray(_CONV_T) @ w2.T).reshape(IMG, N_POS, OUT_CH)
    M = jnp.transpose(M, (0, 2, 1)).reshape(IMG, FLAT)
    mb = jnp.repeat(conv_b, N_POS).reshape(1, FLAT)

    y2 = pl.pallas_call(
        _mlp_kernel,
        out_shape=jax.ShapeDtypeStruct((BS, H_DIM), f32),
    )(img, M, mb, fc_w, fc_b.reshape(1, H_DIM))

    bias_pad = jnp.zeros((1, N_ENT_PAD), f32).at[0, :N_ENT].set(ent_bias)
    blk = 2048
    probs = pl.pallas_call(
        _logits_kernel,
        grid=(N_ENT_PAD // blk,),
        in_specs=[
            pl.BlockSpec((BS, H_DIM), lambda i: (0, 0)),
            pl.BlockSpec((blk, H_DIM), lambda i: (i, 0)),
            pl.BlockSpec((1, blk), lambda i: (0, i)),
        ],
        out_specs=pl.BlockSpec((BS, blk), lambda i: (0, i)),
        out_shape=jax.ShapeDtypeStruct((BS, N_ENT_PAD), f32),
    )(y2, x_pad, bias_pad)
    return probs[:, :N_ENT]


def kernel(h_id, r_id, edge_index, rel_id, ent_emb, rel_emb_1, rel_emb_2,
           W_edge_1, W_node_1, W_comp_1, W_edge_2, W_node_2, W_comp_2,
           pred_rel_emb, conv_w, conv_b, fc_w, fc_b, ent_bias):
    ei = edge_index.astype(i32)
    rid = rel_id.astype(i32)
    x = jnp.zeros((X_ROWS, H_DIM), f32).at[:N_ENT].set(ent_emb)
    rel1 = jnp.zeros((N_RELS_PAD, H_DIM), f32).at[:N_RELS].set(rel_emb_1)
    rel2 = jnp.zeros((N_RELS_PAD, H_DIM), f32).at[:N_RELS].set(rel_emb_2)
    for rel_tbl, We, Wn, Wc in ((rel1, W_edge_1, W_node_1, W_comp_1),
                                (rel2, W_edge_2, W_node_2, W_comp_2)):
        exe, exn, exc, dene, denn, denc = _pass1(x, rel_tbl, ei, rid)
        dene, denn, denc = _den_combine(dene, denn, denc, exe, exn, exc)
        acc_e, acc_n, acc_c = _pass2(
            x, rel_tbl, ei, rid,
            exe.reshape(N_CHUNKS2, CHUNK2), exn.reshape(N_CHUNKS2, CHUNK2),
            exc.reshape(N_CHUNKS2, CHUNK2), dene, denn, denc)
        xc = _combine(x[:N_ENT_PAD], acc_e, acc_n, acc_c, We, Wn, Wc)
        x = jnp.zeros((X_ROWS, H_DIM), f32).at[:N_ENT_PAD].set(xc)
    return _predictor(x, h_id.astype(i32), r_id.astype(i32), pred_rel_emb,
                      conv_w, conv_b, fc_w, fc_b, ent_bias)


# speedup vs baseline: 11.3179x; 1.4329x over previous
import functools

import jax
import jax.numpy as jnp
import numpy as np
from jax import lax
from jax.experimental import pallas as pl
from jax.experimental.pallas import tpu as pltpu
from jax.experimental.pallas import tpu_sc as plsc

N_ENT = 10000
N_REL = 237
N_EDGE = 320000
H_DIM = 128
BS = 1024
OUT_CH = 32
KER = 7
K_H = 8
K_W = 16
FLAT = OUT_CH * (2 * K_H - KER + 1) * (K_W - KER + 1)
N_POS = (2 * K_H - KER + 1) * (K_W - KER + 1)
IMG = 2 * K_H * K_W
N_RELS = 2 * N_REL
N_RELS_PAD = 480

N_ENT_PAD = 10240
X_ROWS = 16384
NW = 32
CHUNK = 128
N_CHUNKS = N_EDGE // CHUNK
CHUNKS_PER_W = -(-N_CHUNKS // NW)
CHUNK2 = 128
N_CHUNKS2 = N_EDGE // CHUNK2
CHUNKS2_PER_W = -(-N_CHUNKS2 // NW)
ENT_SLICE = N_ENT_PAD // 16

_MESH = dict(core_axis_name="c", subcore_axis_name="s")

f32 = jnp.float32
i32 = jnp.int32


def _iota16():
    return lax.broadcasted_iota(i32, (16,), 0)


def _pass1_body(x_hbm, rel_hbm, ei_hbm, rid_hbm,
                ex_e_hbm, ex_n_hbm, ex_c_hbm,
                den_e_hbm, den_n_hbm, den_c_hbm,
                rel_loc, hdst, hsrc, dstv, srcv, relv,
                exe_v, exn_v, exc_v, zb,
                den_e_sh, den_n_sh, den_c_sh, sem1, sem2, sem3):
    c = lax.axis_index("c")
    s = lax.axis_index("s")
    wid = s * 2 + c
    base = s * ENT_SLICE

    def zb_body(i, _):
        zb[pl.ds(i * 16, 16)] = jnp.zeros((16,), f32)
        return 0
    lax.fori_loop(0, ENT_SLICE // 16, zb_body, 0)
    pltpu.sync_copy(zb, den_e_sh.at[pl.ds(base, ENT_SLICE)])
    pltpu.sync_copy(zb, den_n_sh.at[pl.ds(base, ENT_SLICE)])
    pltpu.sync_copy(zb, den_c_sh.at[pl.ds(base, ENT_SLICE)])
    pltpu.sync_copy(rel_hbm, rel_loc)
    plsc.subcore_barrier()

    def chunk_body(k, _):
        cidx = k * NW + wid

        @pl.when(cidx < N_CHUNKS)
        def _():
            off = cidx * CHUNK
            cpd = pltpu.async_copy(ei_hbm.at[1, pl.ds(off, CHUNK)],
                                   dstv, sem3)
            cps = pltpu.async_copy(ei_hbm.at[0, pl.ds(off, CHUNK)],
                                   srcv, sem3)
            cpr = pltpu.async_copy(rid_hbm.at[pl.ds(off, CHUNK)],
                                   relv, sem3)
            cpd.wait()
            cps.wait()
            cpr.wait()
            cp1 = pltpu.async_copy(x_hbm.at[dstv], hdst, sem1)
            cp2 = pltpu.async_copy(x_hbm.at[srcv], hsrc, sem2)
            cp1.wait()
            cp2.wait()

            def g_body(g, _):
                gb = g * 16
                rel16 = relv[pl.ds(gb, 16)]
                se = sn = sc = jnp.zeros((16,), f32)
                for lane in range(16):
                    e = gb + lane
                    r = rel16[lane]
                    ae = an = ac = jnp.zeros((16,), f32)
                    for k2 in range(H_DIM // 16):
                        o = k2 * 16
                        d = hdst[e, pl.ds(o, 16)]
                        s_ = hsrc[e, pl.ds(o, 16)]
                        re_ = rel_loc[r, pl.ds(o, 16)]
                        ae = ae + re_ * d
                        an = an + s_ * d
                        ac = ac + s_ * re_ * d
                    lm = _iota16() == lane
                    se = jnp.where(lm, jnp.sum(ae), se)
                    sn = jnp.where(lm, jnp.sum(an), sn)
                    sc = jnp.where(lm, jnp.sum(ac), sc)
                exe_v[pl.ds(gb, 16)] = jnp.exp(se)
                exn_v[pl.ds(gb, 16)] = jnp.exp(sn)
                exc_v[pl.ds(gb, 16)] = jnp.exp(sc)
                return 0

            lax.fori_loop(0, CHUNK // 16, g_body, 0)
            pltpu.sync_copy(exe_v, ex_e_hbm.at[cidx])
            pltpu.sync_copy(exn_v, ex_n_hbm.at[cidx])
            pltpu.sync_copy(exc_v, ex_c_hbm.at[cidx])
            pltpu.sync_copy(exe_v, den_e_sh.at[dstv], add=True)
            pltpu.sync_copy(exn_v, den_n_sh.at[dstv], add=True)
            pltpu.sync_copy(exc_v, den_c_sh.at[dstv], add=True)
        return 0

    lax.fori_loop(0, CHUNKS_PER_W, chunk_body, 0)
    plsc.subcore_barrier()
    ob = c * N_ENT_PAD + base
    pltpu.sync_copy(den_e_sh.at[pl.ds(base, ENT_SLICE)],
                    den_e_hbm.at[pl.ds(ob, ENT_SLICE)])
    pltpu.sync_copy(den_n_sh.at[pl.ds(base, ENT_SLICE)],
                    den_n_hbm.at[pl.ds(ob, ENT_SLICE)])
    pltpu.sync_copy(den_c_sh.at[pl.ds(base, ENT_SLICE)],
                    den_c_hbm.at[pl.ds(ob, ENT_SLICE)])


def _pass1(x_pad, rel_tbl, ei, rid):
    out_type = (
        [jax.ShapeDtypeStruct((N_CHUNKS, CHUNK), f32)] * 3
        + [jax.ShapeDtypeStruct((2 * N_ENT_PAD,), f32)] * 3)
    scratch = [
        pltpu.VMEM((N_RELS_PAD, H_DIM), f32),
        pltpu.VMEM((CHUNK, H_DIM), f32),
        pltpu.VMEM((CHUNK, H_DIM), f32),
        pltpu.VMEM((CHUNK,), i32),
        pltpu.VMEM((CHUNK,), i32),
        pltpu.VMEM((CHUNK,), i32),
        pltpu.VMEM((CHUNK,), f32),
        pltpu.VMEM((CHUNK,), f32),
        pltpu.VMEM((CHUNK,), f32),
        pltpu.VMEM((ENT_SLICE,), f32),
        pltpu.VMEM_SHARED((N_ENT_PAD,), f32),
        pltpu.VMEM_SHARED((N_ENT_PAD,), f32),
        pltpu.VMEM_SHARED((N_ENT_PAD,), f32),
        pltpu.SemaphoreType.DMA,
        pltpu.SemaphoreType.DMA,
        pltpu.SemaphoreType.DMA,
    ]
    return pl.kernel(
        _pass1_body, out_type=out_type,
        mesh=plsc.VectorSubcoreMesh(**_MESH),
        compiler_params=pltpu.CompilerParams(needs_layout_passes=False),
        scratch_types=scratch,
    )(x_pad, rel_tbl, ei, rid)


def _pass2_body(x_hbm, rel_hbm, ei_hbm, rid_hbm,
                exe_hbm, exn_hbm, exc_hbm,
                dene_hbm, denn_hbm, denc_hbm,
                oute_hbm, outn_hbm, outc_hbm,
                re_b, hs, dstv, srcv, relv, exv, denv, zb,
                acc_sh, sem1, sem2, sem3, sem4):
    c = lax.axis_index("c")
    s = lax.axis_index("s")
    wid = s * 2 + c
    base = s * ENT_SLICE

    for r in range(16):
        def zb_body(j, _):
            zb[r, pl.ds(j * 16, 16)] = jnp.zeros((16,), f32)
            return 0
        lax.fori_loop(0, H_DIM // 16, zb_body, 0)

    for mode, ex_hbm, den_hbm, out_hbm in (
            (0, exe_hbm, dene_hbm, oute_hbm),
            (1, exn_hbm, denn_hbm, outn_hbm),
            (2, exc_hbm, denc_hbm, outc_hbm)):
        need_rel = mode in (0, 2)
        need_src = mode in (1, 2)
        for t in range(ENT_SLICE // 16):
            pltpu.sync_copy(zb, acc_sh.at[pl.ds(base + t * 16, 16)])
        pltpu.sync_copy(den_hbm, denv)
        plsc.subcore_barrier()

        def chunk_body(k, _):
            cidx = k * NW + wid

            @pl.when(cidx < N_CHUNKS2)
            def _():
                srcbuf = hs if mode == 1 else re_b

                @pl.when(k > 0)
                def _():
                    pltpu.make_async_copy(srcbuf, acc_sh.at[dstv],
                                          sem4).wait()

                off = cidx * CHUNK2
                cpd = pltpu.async_copy(ei_hbm.at[1, pl.ds(off, CHUNK2)],
                                       dstv, sem3)
                cpe = pltpu.async_copy(ex_hbm.at[cidx], exv, sem3)
                if need_rel:
                    cpr = pltpu.async_copy(rid_hbm.at[pl.ds(off, CHUNK2)],
                                           relv, sem3)
                if need_src:
                    cps = pltpu.async_copy(ei_hbm.at[0, pl.ds(off, CHUNK2)],
                                           srcv, sem3)
                cpd.wait()
                cpe.wait()
                if need_rel:
                    cpr.wait()
                    cp2 = pltpu.async_copy(rel_hbm.at[relv], re_b, sem2)
                if need_src:
                    cps.wait()
                    cp1 = pltpu.async_copy(x_hbm.at[srcv], hs, sem1)
                if need_rel:
                    cp2.wait()
                if need_src:
                    cp1.wait()

                def g_body(g, _):
                    gb = g * 16
                    dst16 = dstv[pl.ds(gb, 16)]
                    d16 = plsc.load_gather(denv, [dst16])
                    a16 = exv[pl.ds(gb, 16)] / d16
                    for lane in range(16):
                        e = gb + lane
                        a_s = a16[lane]
                        for k2 in range(H_DIM // 16):
                            o = k2 * 16
                            if mode == 0:
                                re_b[e, pl.ds(o, 16)] = (
                                    a_s * re_b[e, pl.ds(o, 16)])
                            elif mode == 1:
                                hs[e, pl.ds(o, 16)] = (
                                    a_s * hs[e, pl.ds(o, 16)])
                            else:
                                re_b[e, pl.ds(o, 16)] = (
                                    a_s * re_b[e, pl.ds(o, 16)]
                                    * hs[e, pl.ds(o, 16)])
                    return 0

                lax.fori_loop(0, CHUNK2 // 16, g_body, 0)
                pltpu.async_copy(srcbuf, acc_sh.at[dstv], sem4, add=True)
            return 0

        lax.fori_loop(0, CHUNKS2_PER_W, chunk_body, 0)
        drainbuf = hs if mode == 1 else re_b
        pltpu.make_async_copy(drainbuf, acc_sh.at[dstv], sem4).wait()
        plsc.subcore_barrier()
        ob = c * N_ENT_PAD + base
        for t in range(ENT_SLICE // CHUNK):
            pltpu.sync_copy(acc_sh.at[pl.ds(base + t * CHUNK, CHUNK)],
                            out_hbm.at[pl.ds(ob + t * CHUNK, CHUNK)])


def _pass2(x_pad, rel_tbl, ei, rid, exe, exn, exc, dene, denn, denc):
    out_type = [jax.ShapeDtypeStruct((2 * N_ENT_PAD, H_DIM), f32)] * 3
    scratch = [
        pltpu.VMEM((CHUNK2, H_DIM), f32),
        pltpu.VMEM((CHUNK2, H_DIM), f32),
        pltpu.VMEM((CHUNK2,), i32),
        pltpu.VMEM((CHUNK2,), i32),
        pltpu.VMEM((CHUNK2,), i32),
        pltpu.VMEM((CHUNK2,), f32),
        pltpu.VMEM((N_ENT_PAD,), f32),
        pltpu.VMEM((16, H_DIM), f32),
        pltpu.VMEM_SHARED((N_ENT_PAD, H_DIM), f32),
        pltpu.SemaphoreType.DMA,
        pltpu.SemaphoreType.DMA,
        pltpu.SemaphoreType.DMA,
        pltpu.SemaphoreType.DMA,
    ]
    outs = pl.kernel(
        _pass2_body, out_type=out_type,
        mesh=plsc.VectorSubcoreMesh(**_MESH),
        compiler_params=pltpu.CompilerParams(needs_layout_passes=False),
        scratch_types=scratch,
    )(x_pad, rel_tbl, ei, rid, exe, exn, exc, dene, denn, denc)
    return tuple(o.reshape(2, N_ENT_PAD, H_DIM) for o in outs)


def _dencomb_kernel(d_ref, exe_ref, exn_ref, exc_ref, out_ref):
    out_ref[...] = (d_ref[:, 0, :] + d_ref[:, 1, :]
                    + exe_ref[0, 0] * 0.0 + exn_ref[0, 0] * 0.0
                    + exc_ref[0, 0] * 0.0)


def _den_combine(dene, denn, denc, exe, exn, exc):
    d = jnp.stack([dene, denn, denc]).reshape(3, 2, N_ENT_PAD)
    out = pl.pallas_call(
        _dencomb_kernel,
        out_shape=jax.ShapeDtypeStruct((3, N_ENT_PAD), f32),
    )(d, exe, exn, exc)
    return out[0], out[1], out[2]


def _combine_kernel(x_ref, ae_ref, an_ref, ac_ref, we_ref, wn_ref, wc_ref,
                    out_ref):
    ae = ae_ref[0] + ae_ref[1]
    an = an_ref[0] + an_ref[1]
    ac = ac_ref[0] + ac_ref[1]
    out_ref[...] = (
        x_ref[...]
        + jnp.tanh(jnp.dot(ae, we_ref[...], preferred_element_type=f32))
        + jnp.tanh(jnp.dot(an, wn_ref[...], preferred_element_type=f32))
        + jnp.tanh(jnp.dot(ac, wc_ref[...], preferred_element_type=f32)))


def _combine(x_pad, acc_e, acc_n, acc_c, We, Wn, Wc):
    blk = 1024
    acc_spec = pl.BlockSpec((2, blk, H_DIM), lambda i: (0, i, 0))
    w_spec = pl.BlockSpec((H_DIM, H_DIM), lambda i: (0, 0))
    return pl.pallas_call(
        _combine_kernel,
        grid=(N_ENT_PAD // blk,),
        in_specs=[pl.BlockSpec((blk, H_DIM), lambda i: (i, 0)),
                  acc_spec, acc_spec, acc_spec, w_spec, w_spec, w_spec],
        out_specs=pl.BlockSpec((blk, H_DIM), lambda i: (i, 0)),
        out_shape=jax.ShapeDtypeStruct((N_ENT_PAD, H_DIM), f32),
    )(x_pad, acc_e, acc_n, acc_c, We, Wn, Wc)


def _gather_body(tbl, ix, out, ixv, rows, sem):
    per = ixv.shape[0]
    wid = lax.axis_index("s") * 2 + lax.axis_index("c")
    base = wid * per
    pltpu.sync_copy(ix.at[pl.ds(base, per)], ixv)
    pltpu.async_copy(tbl.at[ixv], rows, sem).wait()
    pltpu.sync_copy(rows, out.at[pl.ds(base, per)])


def _gather_rows(table, idx):
    B = idx.shape[0]
    per = B // NW
    return pl.kernel(
        _gather_body,
        out_type=jax.ShapeDtypeStruct((B, table.shape[1]), f32),
        mesh=plsc.VectorSubcoreMesh(**_MESH),
        compiler_params=pltpu.CompilerParams(needs_layout_passes=False),
        scratch_types=[pltpu.VMEM((per,), i32),
                       pltpu.VMEM((per, table.shape[1]), f32),
                       pltpu.SemaphoreType.DMA],
    )(table, idx)


def _conv_as_matmul_tensor():
    T = np.zeros((IMG, N_POS, KER * KER), np.float32)
    for i in range(10):
        for j in range(10):
            for kh in range(KER):
                for kw in range(KER):
                    T[K_W * (i + kh) + (j + kw), i * 10 + j,
                      kh * KER + kw] = 1.0
    return T.reshape(IMG * N_POS, KER * KER)


_CONV_T = _conv_as_matmul_tensor()


def _mlp_kernel(img_ref, m_ref, mb_ref, fcw_ref, fcb_ref, out_ref):
    y1 = jnp.maximum(
        jnp.dot(img_ref[...], m_ref[...], preferred_element_type=f32)
        + mb_ref[...], 0.0)
    y2 = jnp.maximum(
        jnp.dot(y1, fcw_ref[...], preferred_element_type=f32)
        + fcb_ref[...], 0.0)
    out_ref[...] = y2


def _logits_kernel(y_ref, x_ref, b_ref, out_ref):
    l = lax.dot_general(y_ref[...], x_ref[...],
                        (((1,), (1,)), ((), ())), preferred_element_type=f32)
    out_ref[...] = jax.nn.sigmoid(l + b_ref[...])


def _predictor(x_big, h_id, r_id, pred_rel_emb, conv_w, conv_b, fc_w, fc_b,
               ent_bias):
    x_pad = x_big[:N_ENT_PAD]
    head = _gather_rows(x_big, h_id)
    rel = _gather_rows(pred_rel_emb, r_id)
    img = jnp.stack([head, rel], axis=2).reshape(BS, IMG)

    w2 = conv_w.reshape(OUT_CH, KER * KER)
    M = (jnp.asarray(_CONV_T) @ w2.T).reshape(IMG, N_POS, OUT_CH)
    M = jnp.transpose(M, (0, 2, 1)).reshape(IMG, FLAT)
    mb = jnp.repeat(conv_b, N_POS).reshape(1, FLAT)

    y2 = pl.pallas_call(
        _mlp_kernel,
        out_shape=jax.ShapeDtypeStruct((BS, H_DIM), f32),
    )(img, M, mb, fc_w, fc_b.reshape(1, H_DIM))

    bias_pad = jnp.zeros((1, N_ENT_PAD), f32).at[0, :N_ENT].set(ent_bias)
    blk = 2048
    probs = pl.pallas_call(
        _logits_kernel,
        grid=(N_ENT_PAD // blk,),
        in_specs=[
            pl.BlockSpec((BS, H_DIM), lambda i: (0, 0)),
            pl.BlockSpec((blk, H_DIM), lambda i: (i, 0)),
            pl.BlockSpec((1, blk), lambda i: (0, i)),
        ],
        out_specs=pl.BlockSpec((BS, blk), lambda i: (0, i)),
        out_shape=jax.ShapeDtypeStruct((BS, N_ENT_PAD), f32),
    )(y2, x_pad, bias_pad)
    return probs[:, :N_ENT]


def kernel(h_id, r_id, edge_index, rel_id, ent_emb, rel_emb_1, rel_emb_2,
           W_edge_1, W_node_1, W_comp_1, W_edge_2, W_node_2, W_comp_2,
           pred_rel_emb, conv_w, conv_b, fc_w, fc_b, ent_bias):
    ei = edge_index.astype(i32)
    rid = rel_id.astype(i32)
    x = jnp.zeros((X_ROWS, H_DIM), f32).at[:N_ENT].set(ent_emb)
    rel1 = jnp.zeros((N_RELS_PAD, H_DIM), f32).at[:N_RELS].set(rel_emb_1)
    rel2 = jnp.zeros((N_RELS_PAD, H_DIM), f32).at[:N_RELS].set(rel_emb_2)
    for rel_tbl, We, Wn, Wc in ((rel1, W_edge_1, W_node_1, W_comp_1),
                                (rel2, W_edge_2, W_node_2, W_comp_2)):
        exe, exn, exc, dene, denn, denc = _pass1(x, rel_tbl, ei, rid)
        dene, denn, denc = _den_combine(dene, denn, denc, exe, exn, exc)
        acc_e, acc_n, acc_c = _pass2(
            x, rel_tbl, ei, rid,
            exe.reshape(N_CHUNKS2, CHUNK2), exn.reshape(N_CHUNKS2, CHUNK2),
            exc.reshape(N_CHUNKS2, CHUNK2), dene, denn, denc)
        xc = _combine(x[:N_ENT_PAD], acc_e, acc_n, acc_c, We, Wn, Wc)
        x = jnp.zeros((X_ROWS, H_DIM), f32).at[:N_ENT_PAD].set(xc)
    return _predictor(x, h_id.astype(i32), r_id.astype(i32), pred_rel_emb,
                      conv_w, conv_b, fc_w, fc_b, ent_bias)
